# Initial kernel scaffold; baseline (speedup 1.0000x reference)
import jax
import jax.numpy as jnp
from jax.experimental import pallas as pl


def kernel(tokens, attention_x, attention_y):
    raise NotImplementedError("write your pallas kernel here")


# speedup vs baseline: 2.6584x; 2.6584x over previous
import functools
import math

import jax
import jax.numpy as jnp
from jax import lax
from jax.experimental import pallas as pl
from jax.experimental.pallas import tpu as pltpu
from jax.experimental.pallas import tpu_sc as plsc

_NC = 2
_NS = 16
_NW = _NC * _NS

_CH = 256
_JB = 2048
_KC = 1024
_GCH = 64


def _total_order_key(s):
    a = lax.bitcast_convert_type(s, jnp.int32)
    return jnp.where(a < 0, a ^ jnp.int32(0x7FFFFFFF), a)


def _rank_body(x_ref, y_ref, xc_ref, yc_ref, rank_ref, *, L):
    i = pl.program_id(1)
    kr = _total_order_key(x_ref[0] + y_ref[0])
    kc = _total_order_key(xc_ref[0] + yc_ref[0])
    icol = i * _CH + lax.broadcasted_iota(jnp.int32, (_CH, 1), 0)
    acc = jnp.zeros((_CH, 1), jnp.float32)
    for jb in range(L // _JB):
        krb = lax.slice(kr, (0, jb * _JB), (1, (jb + 1) * _JB))
        jrow = jb * _JB + lax.broadcasted_iota(jnp.int32, (1, _JB), 1)
        gt = krb > kc
        tie = (krb == kc) & (jrow < icol)
        cnt = jnp.where(gt | tie, 1.0, 0.0)
        acc = acc + jnp.sum(cnt, axis=1, keepdims=True)
    rank_ref[0] = acc.astype(jnp.int32)


def _extra_body(x_ref, y_ref, rank_ref, tok_ref, out_ref, w_ref, *, K):
    k = pl.program_id(1)

    @pl.when(k == 0)
    def _():
        s = x_ref[0] + y_ref[0]
        tail = jnp.where(rank_ref[0] >= K, s, -jnp.inf)
        m = jnp.max(tail)
        e = jnp.exp(tail - m)
        w_ref[...] = e / jnp.sum(e)

    wk = w_ref[pl.ds(k * _KC, _KC), :]
    part = lax.dot_general(wk, tok_ref[0], (((0,), (0,)), ((), ())),
                           preferred_element_type=jnp.float32)

    @pl.when(k == 0)
    def _():
        out_ref[0] = part

    @pl.when(k > 0)
    def _():
        out_ref[0] += part


def _make_gather(B, L, C, K):
    n_out = B * K
    n_ch = -(-n_out // _GCH)
    rounds = -(-n_ch // _NW)
    last_start = n_out - _GCH
    assert last_start % 8 == 0 and n_out % 8 == 0
    mesh = plsc.VectorSubcoreMesh(core_axis_name="c", subcore_axis_name="s")

    @functools.partial(
        pl.kernel,
        out_type=jax.ShapeDtypeStruct((n_out, C), jnp.float32),
        mesh=mesh,
        compiler_params=pltpu.CompilerParams(needs_layout_passes=False),
        scratch_types=[
            pltpu.VMEM((B * L,), jnp.int32),
            pltpu.VMEM((B * L,), jnp.int32),
            pltpu.VMEM((_GCH,), jnp.int32),
            pltpu.VMEM((_GCH, C), jnp.float32),
            pltpu.SemaphoreType.DMA,
        ],
    )
    def gather(rank_hbm, tok_hbm, out_hbm, rank_v, inv_v, gidx_v, rows_v, sem):
        wid = lax.axis_index("s") * _NC + lax.axis_index("c")
        pltpu.sync_copy(rank_hbm, rank_v)

        def build(i, carry):
            r16 = rank_v[pl.ds(i * 16, 16)]
            base = (i // (L // 16)) * L
            plsc.store_scatter(inv_v, [r16 + base],
                               lax.iota(jnp.int32, 16) + i * 16)
            return carry

        lax.fori_loop(0, B * L // 16, build, jnp.int32(0))

        for cc in range(rounds):
            start = jnp.minimum((wid + cc * _NW) * _GCH, last_start)
            for j in range(_GCH // 16):
                pos = lax.iota(jnp.int32, 16) + (start + j * 16)
                bsel = pos // K
                gidx_v[pl.ds(j * 16, 16)] = plsc.load_gather(
                    inv_v, [pos - bsel * K + bsel * L])
            pltpu.async_copy(tok_hbm.at[gidx_v], rows_v, sem).wait()
            pltpu.sync_copy(rows_v, out_hbm.at[pl.ds(start, _GCH)])

    return gather


def kernel(tokens, attention_x, attention_y):
    B, L, C = tokens.shape
    K = math.ceil(L * 0.6)

    rank3 = pl.pallas_call(
        functools.partial(_rank_body, L=L),
        grid=(B, L // _CH),
        in_specs=[
            pl.BlockSpec((1, 1, L), lambda b, i: (b, 0, 0)),
            pl.BlockSpec((1, 1, L), lambda b, i: (b, 0, 0)),
            pl.BlockSpec((1, _CH, 1), lambda b, i: (b, i, 0)),
            pl.BlockSpec((1, _CH, 1), lambda b, i: (b, i, 0)),
        ],
        out_specs=pl.BlockSpec((1, _CH, 1), lambda b, i: (b, i, 0)),
        out_shape=jax.ShapeDtypeStruct((B, L, 1), jnp.int32),
    )(attention_x[:, None, :], attention_y[:, None, :],
      attention_x[:, :, None], attention_y[:, :, None])

    extra = pl.pallas_call(
        functools.partial(_extra_body, K=K),
        grid=(B, L // _KC),
        in_specs=[
            pl.BlockSpec((1, L, 1), lambda b, k: (b, 0, 0)),
            pl.BlockSpec((1, L, 1), lambda b, k: (b, 0, 0)),
            pl.BlockSpec((1, L, 1), lambda b, k: (b, 0, 0)),
            pl.BlockSpec((1, _KC, C), lambda b, k: (b, k, 0)),
        ],
        out_specs=pl.BlockSpec((1, 1, C), lambda b, k: (b, 0, 0)),
        out_shape=jax.ShapeDtypeStruct((B, 1, C), jnp.float32),
        scratch_shapes=[pltpu.VMEM((L, 1), jnp.float32)],
    )(attention_x[:, :, None], attention_y[:, :, None], rank3, tokens)

    sel = _make_gather(B, L, C, K)(rank3.reshape(B * L), tokens.reshape(B * L, C))
    return sel.reshape(B, K, C), extra


# speedup vs baseline: 2.8008x; 1.0536x over previous
import functools
import math

import jax
import jax.numpy as jnp
from jax import lax
from jax.experimental import pallas as pl
from jax.experimental.pallas import tpu as pltpu
from jax.experimental.pallas import tpu_sc as plsc

_NC = 2
_NS = 16
_NW = _NC * _NS

_CH = 256
_JB = 2048
_KC = 1024
_GCH = 64


def _total_order_key(s):
    a = lax.bitcast_convert_type(s, jnp.int32)
    return jnp.where(a < 0, a ^ jnp.int32(0x7FFFFFFF), a)


def _rank_body(x_ref, y_ref, xc_ref, yc_ref, rank_ref, *, L):
    i = pl.program_id(1)
    kr = _total_order_key(x_ref[0] + y_ref[0])
    kc = _total_order_key(xc_ref[0] + yc_ref[0])
    icol = i * _CH + lax.broadcasted_iota(jnp.int32, (_CH, 1), 0)
    acc = jnp.zeros((_CH, 1), jnp.float32)
    for jb in range(L // _JB):
        krb = lax.slice(kr, (0, jb * _JB), (1, (jb + 1) * _JB))
        jrow = jb * _JB + lax.broadcasted_iota(jnp.int32, (1, _JB), 1)
        gt = krb > kc
        tie = (krb == kc) & (jrow < icol)
        cnt = jnp.where(gt | tie, 1.0, 0.0)
        acc = acc + jnp.sum(cnt, axis=1, keepdims=True)
    rank_ref[0] = acc.astype(jnp.int32)


def _extra_body(x_ref, y_ref, rank_ref, tok_ref, out_ref, w_ref, *, K):
    k = pl.program_id(1)

    @pl.when(k == 0)
    def _():
        s = x_ref[0] + y_ref[0]
        tail = jnp.where(rank_ref[0] >= K, s, -jnp.inf)
        m = jnp.max(tail)
        e = jnp.exp(tail - m)
        w_ref[...] = e / jnp.sum(e)

    wk = w_ref[pl.ds(k * _KC, _KC), :]
    part = lax.dot_general(wk, tok_ref[0], (((0,), (0,)), ((), ())),
                           preferred_element_type=jnp.float32)

    @pl.when(k == 0)
    def _():
        out_ref[0] = part

    @pl.when(k > 0)
    def _():
        out_ref[0] += part


def _make_gather(B, L, C, K, Kp):
    n_out = B * Kp
    n_ch = -(-n_out // _GCH)
    rounds = -(-n_ch // _NW)
    last_start = n_out - _GCH
    assert last_start % 8 == 0
    mesh = plsc.VectorSubcoreMesh(core_axis_name="c", subcore_axis_name="s")

    @functools.partial(
        pl.kernel,
        out_type=jax.ShapeDtypeStruct((n_out, C), jnp.float32),
        mesh=mesh,
        compiler_params=pltpu.CompilerParams(needs_layout_passes=False),
        scratch_types=[
            pltpu.VMEM((B * L,), jnp.int32),
            pltpu.VMEM((B * L,), jnp.int32),
            pltpu.VMEM((_GCH,), jnp.int32),
            pltpu.VMEM((_GCH, C), jnp.float32),
            pltpu.SemaphoreType.DMA,
        ],
    )
    def gather(rank_hbm, tok_hbm, out_hbm, rank_v, inv_v, gidx_v, rows_v, sem):
        wid = lax.axis_index("s") * _NC + lax.axis_index("c")
        pltpu.sync_copy(rank_hbm, rank_v)

        def build(i, carry):
            r16 = rank_v[pl.ds(i * 16, 16)]
            base = (i // (L // 16)) * L
            plsc.store_scatter(inv_v, [r16 + base],
                               lax.iota(jnp.int32, 16) + i * 16)
            return carry

        lax.fori_loop(0, B * L // 16, build, jnp.int32(0))

        for cc in range(rounds):
            start = jnp.minimum((wid + cc * _NW) * _GCH, last_start)
            for j in range(_GCH // 16):
                pos = lax.iota(jnp.int32, 16) + (start + j * 16)
                bsel = pos // Kp
                gidx_v[pl.ds(j * 16, 16)] = plsc.load_gather(
                    inv_v, [pos - bsel * Kp + bsel * L])
            pltpu.async_copy(tok_hbm.at[gidx_v], rows_v, sem).wait()
            pltpu.sync_copy(rows_v, out_hbm.at[pl.ds(start, _GCH)])

    return gather


def kernel(tokens, attention_x, attention_y):
    B, L, C = tokens.shape
    K = math.ceil(L * 0.6)

    rank3 = pl.pallas_call(
        functools.partial(_rank_body, L=L),
        grid=(B, L // _CH),
        in_specs=[
            pl.BlockSpec((1, 1, L), lambda b, i: (b, 0, 0)),
            pl.BlockSpec((1, 1, L), lambda b, i: (b, 0, 0)),
            pl.BlockSpec((1, _CH, 1), lambda b, i: (b, i, 0)),
            pl.BlockSpec((1, _CH, 1), lambda b, i: (b, i, 0)),
        ],
        out_specs=pl.BlockSpec((1, _CH, 1), lambda b, i: (b, i, 0)),
        out_shape=jax.ShapeDtypeStruct((B, L, 1), jnp.int32),
    )(attention_x[:, None, :], attention_y[:, None, :],
      attention_x[:, :, None], attention_y[:, :, None])

    extra = pl.pallas_call(
        functools.partial(_extra_body, K=K),
        grid=(B, L // _KC),
        in_specs=[
            pl.BlockSpec((1, L, 1), lambda b, k: (b, 0, 0)),
            pl.BlockSpec((1, L, 1), lambda b, k: (b, 0, 0)),
            pl.BlockSpec((1, L, 1), lambda b, k: (b, 0, 0)),
            pl.BlockSpec((1, _KC, C), lambda b, k: (b, k, 0)),
        ],
        out_specs=pl.BlockSpec((1, 1, C), lambda b, k: (b, 0, 0)),
        out_shape=jax.ShapeDtypeStruct((B, 1, C), jnp.float32),
        scratch_shapes=[pltpu.VMEM((L, 1), jnp.float32)],
    )(attention_x[:, :, None], attention_y[:, :, None], rank3, tokens)

    Kp = -(-K // 8) * 8
    sel = _make_gather(B, L, C, K, Kp)(
        rank3.reshape(B * L), tokens.reshape(B * L, C))
    sel = sel.reshape(B, Kp, C)[:, :K]
    return sel, extra


# speedup vs baseline: 4.3269x; 1.5449x over previous
import functools
import math

import jax
import jax.numpy as jnp
from jax import lax
from jax.experimental import pallas as pl
from jax.experimental.pallas import tpu as pltpu
from jax.experimental.pallas import tpu_sc as plsc

_NC = 2
_NS = 16
_NW = _NC * _NS

_CH = 256
_JB = 2048
_KC = 1024
_GCH = 64


def _total_order_key(s):
    a = lax.bitcast_convert_type(s, jnp.int32)
    return jnp.where(a < 0, a ^ jnp.int32(0x7FFFFFFF), a)


def _rank_body(x_ref, y_ref, xc_ref, yc_ref, rank_ref, *, L):
    i = pl.program_id(1)
    kr = _total_order_key(x_ref[0] + y_ref[0])
    kc = _total_order_key(xc_ref[0] + yc_ref[0])
    icol = i * _CH + lax.broadcasted_iota(jnp.int32, (_CH, 1), 0)
    acc = jnp.zeros((_CH, 1), jnp.float32)
    for jb in range(L // _JB):
        krb = lax.slice(kr, (0, jb * _JB), (1, (jb + 1) * _JB))
        jrow = jb * _JB + lax.broadcasted_iota(jnp.int32, (1, _JB), 1)
        gt = krb > kc
        tie = (krb == kc) & (jrow < icol)
        cnt = jnp.where(gt | tie, 1.0, 0.0)
        acc = acc + jnp.sum(cnt, axis=1, keepdims=True)
    rank_ref[0] = acc.astype(jnp.int32)


def _extra_body(x_ref, y_ref, rank_ref, tok_ref, out_ref, w_ref, *, K):
    k = pl.program_id(1)

    @pl.when(k == 0)
    def _():
        s = x_ref[0] + y_ref[0]
        tail = jnp.where(rank_ref[0] >= K, s, -jnp.inf)
        m = jnp.max(tail)
        e = jnp.exp(tail - m)
        w_ref[...] = e / jnp.sum(e)

    wk = w_ref[pl.ds(k * _KC, _KC), :]
    part = lax.dot_general(wk, tok_ref[0], (((0,), (0,)), ((), ())),
                           preferred_element_type=jnp.float32)

    @pl.when(k == 0)
    def _():
        out_ref[0] = part

    @pl.when(k > 0)
    def _():
        out_ref[0] += part


def _make_gather(B, L, C, K, Kp):
    n_out = B * Kp
    n_ch = -(-n_out // _GCH)
    rounds = -(-n_ch // _NW)
    last_start = n_out - _GCH
    assert last_start % 8 == 0
    mesh = plsc.VectorSubcoreMesh(core_axis_name="c", subcore_axis_name="s")

    @functools.partial(
        pl.kernel,
        out_type=jax.ShapeDtypeStruct((n_out, C), jnp.float32),
        mesh=mesh,
        compiler_params=pltpu.CompilerParams(needs_layout_passes=False),
        scratch_types=[
            pltpu.VMEM((B * L,), jnp.int32),
            pltpu.VMEM((B * L,), jnp.int32),
            pltpu.VMEM((_GCH,), jnp.int32),
            pltpu.VMEM((_GCH, C), jnp.float32),
            pltpu.SemaphoreType.DMA,
        ],
    )
    def gather(rank_hbm, tok_hbm, out_hbm, rank_v, inv_v, gidx_v, rows_v, sem):
        wid = lax.axis_index("s") * _NC + lax.axis_index("c")
        pltpu.sync_copy(rank_hbm, rank_v)

        def build(i, carry):
            r16 = rank_v[pl.ds(i * 16, 16)]
            base = (i // (L // 16)) * L
            plsc.store_scatter(inv_v, [r16 + base],
                               lax.iota(jnp.int32, 16) + i * 16)
            return carry

        lax.fori_loop(0, B * L // 16, build, jnp.int32(0))

        for cc in range(rounds):
            start = jnp.minimum((wid + cc * _NW) * _GCH, last_start)
            for j in range(_GCH // 16):
                pos = lax.iota(jnp.int32, 16) + (start + j * 16)
                bsel = pos // Kp
                gidx_v[pl.ds(j * 16, 16)] = plsc.load_gather(
                    inv_v, [pos - bsel * Kp + bsel * L])
            pltpu.async_copy(tok_hbm.at[gidx_v], rows_v, sem).wait()
            pltpu.sync_copy(rows_v, out_hbm.at[pl.ds(start, _GCH)])

    return gather


def kernel(tokens, attention_x, attention_y):
    B, L, C = tokens.shape
    K = math.ceil(L * 0.6)

    rank3 = pl.pallas_call(
        functools.partial(_rank_body, L=L),
        grid=(B, L // _CH),
        in_specs=[
            pl.BlockSpec((1, 1, L), lambda b, i: (b, 0, 0)),
            pl.BlockSpec((1, 1, L), lambda b, i: (b, 0, 0)),
            pl.BlockSpec((1, _CH, 1), lambda b, i: (b, i, 0)),
            pl.BlockSpec((1, _CH, 1), lambda b, i: (b, i, 0)),
        ],
        out_specs=pl.BlockSpec((1, _CH, 1), lambda b, i: (b, i, 0)),
        out_shape=jax.ShapeDtypeStruct((B, L, 1), jnp.int32),
    )(attention_x[:, None, :], attention_y[:, None, :],
      attention_x[:, :, None], attention_y[:, :, None])

    extra = pl.pallas_call(
        functools.partial(_extra_body, K=K),
        grid=(B, L // _KC),
        in_specs=[
            pl.BlockSpec((1, L, 1), lambda b, k: (b, 0, 0)),
            pl.BlockSpec((1, L, 1), lambda b, k: (b, 0, 0)),
            pl.BlockSpec((1, L, 1), lambda b, k: (b, 0, 0)),
            pl.BlockSpec((1, _KC, C), lambda b, k: (b, k, 0)),
        ],
        out_specs=pl.BlockSpec((1, 1, C), lambda b, k: (b, 0, 0)),
        out_shape=jax.ShapeDtypeStruct((B, 1, C), jnp.float32),
        scratch_shapes=[pltpu.VMEM((L, 1), jnp.float32)],
    )(attention_x[:, :, None], attention_y[:, :, None], rank3, tokens)

    Kp = -(-K // 8) * 8
    sel = _make_gather(B, L, C, K, Kp)(
        rank3.reshape(B * L), tokens.reshape(B * L, C))
    one = lax.optimization_barrier(jnp.ones((), jnp.float32))
    sel = sel.reshape(B, Kp, C)[:, :K] * one
    return sel, extra


# speedup vs baseline: 4.8067x; 1.1109x over previous
import functools
import math

import jax
import jax.numpy as jnp
from jax import lax
from jax.experimental import pallas as pl
from jax.experimental.pallas import tpu as pltpu
from jax.experimental.pallas import tpu_sc as plsc

_NC = 2
_NS = 16
_NW = _NC * _NS

_CH = 256
_JB = 2048
_KC = 1024
_GCH = 64


def _total_order_key(s):
    a = lax.bitcast_convert_type(s, jnp.int32)
    return jnp.where(a < 0, a ^ jnp.int32(0x7FFFFFFF), a)


def _rank_body(x_ref, y_ref, xc_ref, yc_ref, rank_ref, *, L):
    i = pl.program_id(1)
    kr = _total_order_key(x_ref[0] + y_ref[0])
    kc = _total_order_key(xc_ref[0] + yc_ref[0])
    icol = i * _CH + lax.broadcasted_iota(jnp.int32, (_CH, 1), 0)
    acc = jnp.zeros((_CH, 1), jnp.float32)
    for jb in range(L // _JB):
        krb = lax.slice(kr, (0, jb * _JB), (1, (jb + 1) * _JB))

        def gt_sum():
            return jnp.sum(jnp.where(krb > kc, 1.0, 0.0),
                           axis=1, keepdims=True)

        def ge_sum():
            return jnp.sum(jnp.where(krb >= kc, 1.0, 0.0),
                           axis=1, keepdims=True)

        def full_sum():
            jrow = jb * _JB + lax.broadcasted_iota(jnp.int32, (1, _JB), 1)
            tie = (krb == kc) & (jrow < icol)
            return jnp.sum(jnp.where((krb > kc) | tie, 1.0, 0.0),
                           axis=1, keepdims=True)

        above = jb * _JB >= (i + 1) * _CH
        below = (jb + 1) * _JB <= i * _CH
        acc = acc + lax.cond(above, gt_sum,
                             lambda: lax.cond(below, ge_sum, full_sum))
    rank_ref[0] = acc.astype(jnp.int32)


def _extra_body(x_ref, y_ref, rank_ref, tok_ref, out_ref, w_ref, *, K):
    k = pl.program_id(1)

    @pl.when(k == 0)
    def _():
        s = x_ref[0] + y_ref[0]
        tail = jnp.where(rank_ref[0] >= K, s, -jnp.inf)
        m = jnp.max(tail)
        e = jnp.exp(tail - m)
        w_ref[...] = e / jnp.sum(e)

    wk = w_ref[pl.ds(k * _KC, _KC), :]
    part = lax.dot_general(wk, tok_ref[0], (((0,), (0,)), ((), ())),
                           preferred_element_type=jnp.float32)

    @pl.when(k == 0)
    def _():
        out_ref[0] = part

    @pl.when(k > 0)
    def _():
        out_ref[0] += part


def _make_gather(B, L, C, K, Kp):
    n_out = B * Kp
    n_ch = -(-n_out // _GCH)
    rounds = -(-n_ch // _NW)
    last_start = n_out - _GCH
    assert last_start % 8 == 0
    mesh = plsc.VectorSubcoreMesh(core_axis_name="c", subcore_axis_name="s")

    @functools.partial(
        pl.kernel,
        out_type=jax.ShapeDtypeStruct((n_out, C), jnp.float32),
        mesh=mesh,
        compiler_params=pltpu.CompilerParams(needs_layout_passes=False),
        scratch_types=[
            pltpu.VMEM((B * L,), jnp.int32),
            pltpu.VMEM((B * L,), jnp.int32),
            pltpu.VMEM((_GCH,), jnp.int32),
            pltpu.VMEM((_GCH, C), jnp.float32),
            pltpu.SemaphoreType.DMA,
        ],
    )
    def gather(rank_hbm, tok_hbm, out_hbm, rank_v, inv_v, gidx_v, rows_v, sem):
        wid = lax.axis_index("s") * _NC + lax.axis_index("c")
        pltpu.sync_copy(rank_hbm, rank_v)

        def build(i, carry):
            r16 = rank_v[pl.ds(i * 16, 16)]
            base = (i // (L // 16)) * L
            plsc.store_scatter(inv_v, [r16 + base],
                               lax.iota(jnp.int32, 16) + i * 16)
            return carry

        lax.fori_loop(0, B * L // 16, build, jnp.int32(0))

        for cc in range(rounds):
            start = jnp.minimum((wid + cc * _NW) * _GCH, last_start)
            for j in range(_GCH // 16):
                pos = lax.iota(jnp.int32, 16) + (start + j * 16)
                bsel = pos // Kp
                gidx_v[pl.ds(j * 16, 16)] = plsc.load_gather(
                    inv_v, [pos - bsel * Kp + bsel * L])
            pltpu.async_copy(tok_hbm.at[gidx_v], rows_v, sem).wait()
            pltpu.sync_copy(rows_v, out_hbm.at[pl.ds(start, _GCH)])

    return gather


def kernel(tokens, attention_x, attention_y):
    B, L, C = tokens.shape
    K = math.ceil(L * 0.6)

    rank3 = pl.pallas_call(
        functools.partial(_rank_body, L=L),
        grid=(B, L // _CH),
        in_specs=[
            pl.BlockSpec((1, 1, L), lambda b, i: (b, 0, 0)),
            pl.BlockSpec((1, 1, L), lambda b, i: (b, 0, 0)),
            pl.BlockSpec((1, _CH, 1), lambda b, i: (b, i, 0)),
            pl.BlockSpec((1, _CH, 1), lambda b, i: (b, i, 0)),
        ],
        out_specs=pl.BlockSpec((1, _CH, 1), lambda b, i: (b, i, 0)),
        out_shape=jax.ShapeDtypeStruct((B, L, 1), jnp.int32),
    )(attention_x[:, None, :], attention_y[:, None, :],
      attention_x[:, :, None], attention_y[:, :, None])

    extra = pl.pallas_call(
        functools.partial(_extra_body, K=K),
        grid=(B, L // _KC),
        in_specs=[
            pl.BlockSpec((1, L, 1), lambda b, k: (b, 0, 0)),
            pl.BlockSpec((1, L, 1), lambda b, k: (b, 0, 0)),
            pl.BlockSpec((1, L, 1), lambda b, k: (b, 0, 0)),
            pl.BlockSpec((1, _KC, C), lambda b, k: (b, k, 0)),
        ],
        out_specs=pl.BlockSpec((1, 1, C), lambda b, k: (b, 0, 0)),
        out_shape=jax.ShapeDtypeStruct((B, 1, C), jnp.float32),
        scratch_shapes=[pltpu.VMEM((L, 1), jnp.float32)],
    )(attention_x[:, :, None], attention_y[:, :, None], rank3, tokens)

    Kp = -(-K // 8) * 8
    sel = _make_gather(B, L, C, K, Kp)(
        rank3.reshape(B * L), tokens.reshape(B * L, C))
    one = lax.optimization_barrier(jnp.ones((), jnp.float32))
    sel = sel.reshape(B, Kp, C)[:, :K] * one
    return sel, extra


# speedup vs baseline: 5.2446x; 1.0911x over previous
import functools
import math

import jax
import jax.numpy as jnp
from jax import lax
from jax.experimental import pallas as pl
from jax.experimental.pallas import tpu as pltpu
from jax.experimental.pallas import tpu_sc as plsc

_NC = 2
_NS = 16
_NW = _NC * _NS

_CH = 512
_JB = 2048
_KC = 1024
_GCH = 64


def _total_order_key(s):
    a = lax.bitcast_convert_type(s, jnp.int32)
    return jnp.where(a < 0, a ^ jnp.int32(0x7FFFFFFF), a)


def _rank_body(x_ref, y_ref, xc_ref, yc_ref, rank_ref, *, L):
    i = pl.program_id(1)
    kr = _total_order_key(x_ref[0] + y_ref[0])
    kc = _total_order_key(xc_ref[0] + yc_ref[0])
    icol = i * _CH + lax.broadcasted_iota(jnp.int32, (_CH, 1), 0)
    acc = jnp.zeros((_CH, 1), jnp.float32)
    for jb in range(L // _JB):
        krb = lax.slice(kr, (0, jb * _JB), (1, (jb + 1) * _JB))

        def gt_sum():
            return jnp.sum(jnp.where(krb > kc, 1.0, 0.0),
                           axis=1, keepdims=True)

        def ge_sum():
            return jnp.sum(jnp.where(krb >= kc, 1.0, 0.0),
                           axis=1, keepdims=True)

        def full_sum():
            jrow = jb * _JB + lax.broadcasted_iota(jnp.int32, (1, _JB), 1)
            tie = (krb == kc) & (jrow < icol)
            return jnp.sum(jnp.where((krb > kc) | tie, 1.0, 0.0),
                           axis=1, keepdims=True)

        above = jb * _JB >= (i + 1) * _CH
        below = (jb + 1) * _JB <= i * _CH
        acc = acc + lax.cond(above, gt_sum,
                             lambda: lax.cond(below, ge_sum, full_sum))
    rank_ref[0] = acc.astype(jnp.int32)


def _extra_body(x_ref, y_ref, rank_ref, tok_ref, out_ref, w_ref, *, K):
    k = pl.program_id(1)

    @pl.when(k == 0)
    def _():
        s = x_ref[0] + y_ref[0]
        tail = jnp.where(rank_ref[0] >= K, s, -jnp.inf)
        m = jnp.max(tail)
        e = jnp.exp(tail - m)
        w_ref[...] = e / jnp.sum(e)

    wk = w_ref[pl.ds(k * _KC, _KC), :]
    part = lax.dot_general(wk, tok_ref[0], (((0,), (0,)), ((), ())),
                           preferred_element_type=jnp.float32)

    @pl.when(k == 0)
    def _():
        out_ref[0] = part

    @pl.when(k > 0)
    def _():
        out_ref[0] += part


def _make_gather(B, L, C, K, Kp):
    n_out = B * Kp
    n_ch = -(-n_out // _GCH)
    rounds = -(-n_ch // _NW)
    last_start = n_out - _GCH
    assert last_start % 8 == 0
    mesh = plsc.VectorSubcoreMesh(core_axis_name="c", subcore_axis_name="s")

    @functools.partial(
        pl.kernel,
        out_type=jax.ShapeDtypeStruct((n_out, C), jnp.float32),
        mesh=mesh,
        compiler_params=pltpu.CompilerParams(needs_layout_passes=False),
        scratch_types=[
            pltpu.VMEM((B * L,), jnp.int32),
            pltpu.VMEM((B * L,), jnp.int32),
            pltpu.VMEM((_GCH,), jnp.int32),
            pltpu.VMEM((_GCH, C), jnp.float32),
            pltpu.SemaphoreType.DMA,
        ],
    )
    def gather(rank_hbm, tok_hbm, out_hbm, rank_v, inv_v, gidx_v, rows_v, sem):
        wid = lax.axis_index("s") * _NC + lax.axis_index("c")
        pltpu.sync_copy(rank_hbm, rank_v)

        def build(i, carry):
            r16 = rank_v[pl.ds(i * 16, 16)]
            base = (i // (L // 16)) * L
            plsc.store_scatter(inv_v, [r16 + base],
                               lax.iota(jnp.int32, 16) + i * 16)
            return carry

        lax.fori_loop(0, B * L // 16, build, jnp.int32(0))

        for cc in range(rounds):
            start = jnp.minimum((wid + cc * _NW) * _GCH, last_start)
            for j in range(_GCH // 16):
                pos = lax.iota(jnp.int32, 16) + (start + j * 16)
                bsel = pos // Kp
                gidx_v[pl.ds(j * 16, 16)] = plsc.load_gather(
                    inv_v, [pos - bsel * Kp + bsel * L])
            pltpu.async_copy(tok_hbm.at[gidx_v], rows_v, sem).wait()
            pltpu.sync_copy(rows_v, out_hbm.at[pl.ds(start, _GCH)])

    return gather


def kernel(tokens, attention_x, attention_y):
    B, L, C = tokens.shape
    K = math.ceil(L * 0.6)

    rank3 = pl.pallas_call(
        functools.partial(_rank_body, L=L),
        grid=(B, L // _CH),
        in_specs=[
            pl.BlockSpec((1, 1, L), lambda b, i: (b, 0, 0)),
            pl.BlockSpec((1, 1, L), lambda b, i: (b, 0, 0)),
            pl.BlockSpec((1, _CH, 1), lambda b, i: (b, i, 0)),
            pl.BlockSpec((1, _CH, 1), lambda b, i: (b, i, 0)),
        ],
        out_specs=pl.BlockSpec((1, _CH, 1), lambda b, i: (b, i, 0)),
        out_shape=jax.ShapeDtypeStruct((B, L, 1), jnp.int32),
    )(attention_x[:, None, :], attention_y[:, None, :],
      attention_x[:, :, None], attention_y[:, :, None])

    extra = pl.pallas_call(
        functools.partial(_extra_body, K=K),
        grid=(B, L // _KC),
        in_specs=[
            pl.BlockSpec((1, L, 1), lambda b, k: (b, 0, 0)),
            pl.BlockSpec((1, L, 1), lambda b, k: (b, 0, 0)),
            pl.BlockSpec((1, L, 1), lambda b, k: (b, 0, 0)),
            pl.BlockSpec((1, _KC, C), lambda b, k: (b, k, 0)),
        ],
        out_specs=pl.BlockSpec((1, 1, C), lambda b, k: (b, 0, 0)),
        out_shape=jax.ShapeDtypeStruct((B, 1, C), jnp.float32),
        scratch_shapes=[pltpu.VMEM((L, 1), jnp.float32)],
    )(attention_x[:, :, None], attention_y[:, :, None], rank3, tokens)

    Kp = -(-K // 8) * 8
    sel = _make_gather(B, L, C, K, Kp)(
        rank3.reshape(B * L), tokens.reshape(B * L, C))
    one = lax.optimization_barrier(jnp.ones((), jnp.float32))
    sel = sel.reshape(B, Kp, C)[:, :K] * one
    return sel, extra


# speedup vs baseline: 5.3592x; 1.0219x over previous
import functools
import math

import jax
import jax.numpy as jnp
from jax import lax
from jax.experimental import pallas as pl
from jax.experimental.pallas import tpu as pltpu
from jax.experimental.pallas import tpu_sc as plsc

_NC = 2
_NS = 16
_NW = _NC * _NS

_CH = 1024
_JB = 2048
_KC = 1024
_GCH = 64


def _total_order_key(s):
    a = lax.bitcast_convert_type(s, jnp.int32)
    return jnp.where(a < 0, a ^ jnp.int32(0x7FFFFFFF), a)


def _rank_body(x_ref, y_ref, xc_ref, yc_ref, rank_ref, *, L):
    i = pl.program_id(1)
    kr = _total_order_key(x_ref[0] + y_ref[0])
    kc = _total_order_key(xc_ref[0] + yc_ref[0])
    icol = i * _CH + lax.broadcasted_iota(jnp.int32, (_CH, 1), 0)
    acc = jnp.zeros((_CH, 1), jnp.float32)
    for jb in range(L // _JB):
        krb = lax.slice(kr, (0, jb * _JB), (1, (jb + 1) * _JB))

        def gt_sum():
            return jnp.sum(jnp.where(krb > kc, 1.0, 0.0),
                           axis=1, keepdims=True)

        def ge_sum():
            return jnp.sum(jnp.where(krb >= kc, 1.0, 0.0),
                           axis=1, keepdims=True)

        def full_sum():
            jrow = jb * _JB + lax.broadcasted_iota(jnp.int32, (1, _JB), 1)
            tie = (krb == kc) & (jrow < icol)
            return jnp.sum(jnp.where((krb > kc) | tie, 1.0, 0.0),
                           axis=1, keepdims=True)

        above = jb * _JB >= (i + 1) * _CH
        below = (jb + 1) * _JB <= i * _CH
        acc = acc + lax.cond(above, gt_sum,
                             lambda: lax.cond(below, ge_sum, full_sum))
    rank_ref[0] = acc.astype(jnp.int32)


def _extra_body(x_ref, y_ref, rank_ref, tok_ref, out_ref, w_ref, *, K):
    k = pl.program_id(1)

    @pl.when(k == 0)
    def _():
        s = x_ref[0] + y_ref[0]
        tail = jnp.where(rank_ref[0] >= K, s, -jnp.inf)
        m = jnp.max(tail)
        e = jnp.exp(tail - m)
        w_ref[...] = e / jnp.sum(e)

    wk = w_ref[pl.ds(k * _KC, _KC), :]
    part = lax.dot_general(wk, tok_ref[0], (((0,), (0,)), ((), ())),
                           preferred_element_type=jnp.float32)

    @pl.when(k == 0)
    def _():
        out_ref[0] = part

    @pl.when(k > 0)
    def _():
        out_ref[0] += part


def _make_gather(B, L, C, K, Kp):
    n_out = B * Kp
    n_ch = -(-n_out // _GCH)
    rounds = -(-n_ch // _NW)
    last_start = n_out - _GCH
    assert last_start % 8 == 0
    mesh = plsc.VectorSubcoreMesh(core_axis_name="c", subcore_axis_name="s")

    @functools.partial(
        pl.kernel,
        out_type=jax.ShapeDtypeStruct((n_out, C), jnp.float32),
        mesh=mesh,
        compiler_params=pltpu.CompilerParams(needs_layout_passes=False),
        scratch_types=[
            pltpu.VMEM((B * L,), jnp.int32),
            pltpu.VMEM((B * L,), jnp.int32),
            pltpu.VMEM((_GCH,), jnp.int32),
            pltpu.VMEM((_GCH, C), jnp.float32),
            pltpu.SemaphoreType.DMA,
        ],
    )
    def gather(rank_hbm, tok_hbm, out_hbm, rank_v, inv_v, gidx_v, rows_v, sem):
        wid = lax.axis_index("s") * _NC + lax.axis_index("c")
        pltpu.sync_copy(rank_hbm, rank_v)

        def build(i, carry):
            r16 = rank_v[pl.ds(i * 16, 16)]
            base = (i // (L // 16)) * L
            plsc.store_scatter(inv_v, [r16 + base],
                               lax.iota(jnp.int32, 16) + i * 16)
            return carry

        lax.fori_loop(0, B * L // 16, build, jnp.int32(0))

        for cc in range(rounds):
            start = jnp.minimum((wid + cc * _NW) * _GCH, last_start)
            for j in range(_GCH // 16):
                pos = lax.iota(jnp.int32, 16) + (start + j * 16)
                bsel = pos // Kp
                gidx_v[pl.ds(j * 16, 16)] = plsc.load_gather(
                    inv_v, [pos - bsel * Kp + bsel * L])
            pltpu.async_copy(tok_hbm.at[gidx_v], rows_v, sem).wait()
            pltpu.sync_copy(rows_v, out_hbm.at[pl.ds(start, _GCH)])

    return gather


def kernel(tokens, attention_x, attention_y):
    B, L, C = tokens.shape
    K = math.ceil(L * 0.6)

    rank3 = pl.pallas_call(
        functools.partial(_rank_body, L=L),
        grid=(B, L // _CH),
        in_specs=[
            pl.BlockSpec((1, 1, L), lambda b, i: (b, 0, 0)),
            pl.BlockSpec((1, 1, L), lambda b, i: (b, 0, 0)),
            pl.BlockSpec((1, _CH, 1), lambda b, i: (b, i, 0)),
            pl.BlockSpec((1, _CH, 1), lambda b, i: (b, i, 0)),
        ],
        out_specs=pl.BlockSpec((1, _CH, 1), lambda b, i: (b, i, 0)),
        out_shape=jax.ShapeDtypeStruct((B, L, 1), jnp.int32),
    )(attention_x[:, None, :], attention_y[:, None, :],
      attention_x[:, :, None], attention_y[:, :, None])

    extra = pl.pallas_call(
        functools.partial(_extra_body, K=K),
        grid=(B, L // _KC),
        in_specs=[
            pl.BlockSpec((1, L, 1), lambda b, k: (b, 0, 0)),
            pl.BlockSpec((1, L, 1), lambda b, k: (b, 0, 0)),
            pl.BlockSpec((1, L, 1), lambda b, k: (b, 0, 0)),
            pl.BlockSpec((1, _KC, C), lambda b, k: (b, k, 0)),
        ],
        out_specs=pl.BlockSpec((1, 1, C), lambda b, k: (b, 0, 0)),
        out_shape=jax.ShapeDtypeStruct((B, 1, C), jnp.float32),
        scratch_shapes=[pltpu.VMEM((L, 1), jnp.float32)],
    )(attention_x[:, :, None], attention_y[:, :, None], rank3, tokens)

    Kp = -(-K // 8) * 8
    sel = _make_gather(B, L, C, K, Kp)(
        rank3.reshape(B * L), tokens.reshape(B * L, C))
    one = lax.optimization_barrier(jnp.ones((), jnp.float32))
    sel = sel.reshape(B, Kp, C)[:, :K] * one
    return sel, extra
